# Initial kernel scaffold; baseline (speedup 1.0000x reference)
import jax
import jax.numpy as jnp
from jax.experimental import pallas as pl


def kernel(raw_drone, raw_dock, adj, W_pd, b_pd, W_pk, b_pk, W_att, a_att, W_fuse, b_fuse):
    raise NotImplementedError("write your pallas kernel here")


# speedup vs baseline: 1.3157x; 1.3157x over previous
import functools

import jax
import jax.numpy as jnp
from jax.experimental import pallas as pl
from jax.experimental.pallas import tpu as pltpu

_NHEADS = 4
_NHID = 64
_ALPHA = 0.2
_NEG = -9e15


def _elu(x):
    return jnp.where(x > 0, x, jnp.exp(x) - 1.0)


def _dock_prep_kernel(raw_dock_ref, W_pk_ref, b_pk_ref, W_att_ref, A2_ref,
                      h_dock_ref, Whk_ref, sk_ref):
    h_dock = _elu(
        jnp.dot(raw_dock_ref[...], W_pk_ref[...],
                preferred_element_type=jnp.float32) + b_pk_ref[...])
    h_dock_ref[...] = h_dock
    sk_ref[...] = jnp.dot(h_dock, A2_ref[...],
                          preferred_element_type=jnp.float32)
    for h in range(_NHEADS):
        Whk_ref[h] = jnp.dot(h_dock, W_att_ref[h],
                             preferred_element_type=jnp.float32)


def _gat_block_kernel(raw_drone_ref, adj_ref, W_pd_ref, b_pd_ref, A1_ref,
                      skT_ref, Whk_ref, W_fuse_ref, b_fuse_ref, out_ref):
    h_d = _elu(
        jnp.dot(raw_drone_ref[...], W_pd_ref[...],
                preferred_element_type=jnp.float32) + b_pd_ref[...])
    sd = jnp.dot(h_d, A1_ref[...], preferred_element_type=jnp.float32)
    adj = adj_ref[...]
    acc = jnp.broadcast_to(b_fuse_ref[...], out_ref.shape)
    for h in range(_NHEADS):
        e = sd[:, h:h + 1] + skT_ref[h:h + 1, :]
        e = jnp.where(e >= 0, e, _ALPHA * e)
        e = jnp.where(adj > 0, e, _NEG)
        m = jnp.max(e, axis=1, keepdims=True)
        p = jnp.exp(e - m)
        s = jnp.sum(p, axis=1, keepdims=True)
        attn = p / s
        head = _elu(jnp.dot(attn, Whk_ref[h],
                            preferred_element_type=jnp.float32))
        acc = acc + jnp.dot(head, W_fuse_ref[h],
                            preferred_element_type=jnp.float32)
    out_ref[...] = acc


@jax.jit
def kernel(raw_drone, raw_dock, adj, W_pd, b_pd, W_pk, b_pk, W_att, a_att,
           W_fuse, b_fuse):
    n_drone, nfeat_drone = raw_drone.shape
    n_dock = raw_dock.shape[0]
    nheads, nhid, _ = W_att.shape

    A1 = jnp.einsum('hij,hj->ih', W_att, a_att[:, :nhid])
    A2 = jnp.einsum('hij,hj->ih', W_att, a_att[:, nhid:])
    W_fuse_h = W_fuse.reshape(nheads, nhid, nhid)
    b_pk2 = b_pk.reshape(1, nhid)
    b_pd2 = b_pd.reshape(1, nhid)
    b_fuse2 = b_fuse.reshape(1, nhid)

    h_dock, Whk, sk = pl.pallas_call(
        _dock_prep_kernel,
        out_shape=(
            jax.ShapeDtypeStruct((n_dock, nhid), jnp.float32),
            jax.ShapeDtypeStruct((nheads, n_dock, nhid), jnp.float32),
            jax.ShapeDtypeStruct((n_dock, nheads), jnp.float32),
        ),
    )(raw_dock, W_pk, b_pk2, W_att, A2)

    skT = sk.T

    blk = 400
    grid = (n_drone // blk,)
    out_drone = pl.pallas_call(
        _gat_block_kernel,
        grid=grid,
        in_specs=[
            pl.BlockSpec((blk, nfeat_drone), lambda i: (i, 0)),
            pl.BlockSpec((blk, n_dock), lambda i: (i, 0)),
            pl.BlockSpec((nfeat_drone, nhid), lambda i: (0, 0)),
            pl.BlockSpec((1, nhid), lambda i: (0, 0)),
            pl.BlockSpec((nhid, nheads), lambda i: (0, 0)),
            pl.BlockSpec((nheads, n_dock), lambda i: (0, 0)),
            pl.BlockSpec((nheads, n_dock, nhid), lambda i: (0, 0, 0)),
            pl.BlockSpec((nheads, nhid, nhid), lambda i: (0, 0, 0)),
            pl.BlockSpec((1, nhid), lambda i: (0, 0)),
        ],
        out_specs=pl.BlockSpec((blk, nhid), lambda i: (i, 0)),
        out_shape=jax.ShapeDtypeStruct((n_drone, nhid), jnp.float32),
        compiler_params=pltpu.CompilerParams(
            dimension_semantics=("arbitrary",)),
    )(raw_drone, adj, W_pd, b_pd2, A1, skT, Whk, W_fuse_h, b_fuse2)

    return (out_drone, h_dock)


# speedup vs baseline: 1.4109x; 1.0724x over previous
import jax
import jax.numpy as jnp
from jax.experimental import pallas as pl
from jax.experimental.pallas import tpu as pltpu

_NHEADS = 4
_ALPHA = 0.2
_NEG = -9e15


def _elu(x):
    return jnp.where(x > 0, x, jnp.exp(x) - 1.0)


def _dock_prep_kernel(raw_dock_ref, W_pk_ref, b_pk_ref, W_att_ref, A2_ref,
                      h_dock_ref, Whk_ref, sk_ref):
    h_dock = _elu(
        jnp.dot(raw_dock_ref[...], W_pk_ref[...],
                preferred_element_type=jnp.float32) + b_pk_ref[...])
    h_dock_ref[...] = h_dock
    sk_ref[...] = jnp.dot(h_dock, A2_ref[...],
                          preferred_element_type=jnp.float32)
    for h in range(_NHEADS):
        Whk_ref[h] = jnp.dot(h_dock, W_att_ref[h],
                             preferred_element_type=jnp.float32
                             ).astype(jnp.bfloat16)


def _gat_block_kernel(raw_drone_ref, adj_ref, W_pd_ref, b_pd_ref, A1_ref,
                      skT_ref, Whk_ref, W_fuse_ref, b_fuse_ref, out_ref):
    h_d = _elu(
        jnp.dot(raw_drone_ref[...], W_pd_ref[...],
                preferred_element_type=jnp.float32) + b_pd_ref[...])
    sd = jnp.dot(h_d, A1_ref[...], preferred_element_type=jnp.float32)
    adj = adj_ref[...]
    acc = jnp.broadcast_to(b_fuse_ref[...], out_ref.shape)
    for h in range(_NHEADS):
        e = sd[:, h:h + 1] + skT_ref[h:h + 1, :]
        e = jnp.maximum(e, _ALPHA * e)
        e = jnp.where(adj > 0, e, _NEG)
        m = jnp.max(e, axis=1, keepdims=True)
        p32 = jnp.exp(e - m)
        s = jnp.sum(p32, axis=1, keepdims=True)
        p = p32.astype(jnp.bfloat16)
        head_pre = jnp.dot(p, Whk_ref[h],
                           preferred_element_type=jnp.float32)
        head = _elu(head_pre * (1.0 / s))
        acc = acc + jnp.dot(head, W_fuse_ref[h],
                            preferred_element_type=jnp.float32)
    out_ref[...] = acc


@jax.jit
def kernel(raw_drone, raw_dock, adj, W_pd, b_pd, W_pk, b_pk, W_att, a_att,
           W_fuse, b_fuse):
    n_drone, nfeat_drone = raw_drone.shape
    n_dock = raw_dock.shape[0]
    nheads, nhid, _ = W_att.shape

    A1 = jnp.einsum('hij,hj->ih', W_att, a_att[:, :nhid])
    A2 = jnp.einsum('hij,hj->ih', W_att, a_att[:, nhid:])
    W_fuse_h = W_fuse.reshape(nheads, nhid, nhid)
    b_pk2 = b_pk.reshape(1, nhid)
    b_pd2 = b_pd.reshape(1, nhid)
    b_fuse2 = b_fuse.reshape(1, nhid)

    h_dock, Whk, sk = pl.pallas_call(
        _dock_prep_kernel,
        out_shape=(
            jax.ShapeDtypeStruct((n_dock, nhid), jnp.float32),
            jax.ShapeDtypeStruct((nheads, n_dock, nhid), jnp.bfloat16),
            jax.ShapeDtypeStruct((n_dock, nheads), jnp.float32),
        ),
    )(raw_dock, W_pk, b_pk2, W_att, A2)

    skT = sk.T

    blk = 400
    grid = (n_drone // blk,)
    out_drone = pl.pallas_call(
        _gat_block_kernel,
        grid=grid,
        in_specs=[
            pl.BlockSpec((blk, nfeat_drone), lambda i: (i, 0)),
            pl.BlockSpec((blk, n_dock), lambda i: (i, 0)),
            pl.BlockSpec((nfeat_drone, nhid), lambda i: (0, 0)),
            pl.BlockSpec((1, nhid), lambda i: (0, 0)),
            pl.BlockSpec((nhid, nheads), lambda i: (0, 0)),
            pl.BlockSpec((nheads, n_dock), lambda i: (0, 0)),
            pl.BlockSpec((nheads, n_dock, nhid), lambda i: (0, 0, 0)),
            pl.BlockSpec((nheads, nhid, nhid), lambda i: (0, 0, 0)),
            pl.BlockSpec((1, nhid), lambda i: (0, 0)),
        ],
        out_specs=pl.BlockSpec((blk, nhid), lambda i: (i, 0)),
        out_shape=jax.ShapeDtypeStruct((n_drone, nhid), jnp.float32),
        compiler_params=pltpu.CompilerParams(
            dimension_semantics=("arbitrary",)),
    )(raw_drone, adj, W_pd, b_pd2, A1, skT, Whk, W_fuse_h, b_fuse2)

    return (out_drone, h_dock)


# speedup vs baseline: 1.7538x; 1.2430x over previous
import jax
import jax.numpy as jnp
from jax.experimental import pallas as pl
from jax.experimental.pallas import tpu as pltpu

_NHEADS = 4
_NHID = 64
_ALPHA = 0.2
_NEG = -9e15


def _elu(x):
    return jnp.where(x > 0, x, jnp.exp(x) - 1.0)


def _prep_kernel(raw_drone_ref, raw_dock_ref, W_pd_ref, b_pd_ref,
                 W_pk_ref, b_pk_ref, W_att_ref, A1_ref, A2_ref,
                 h_dock_ref, Whk_ref, skT_ref, sd_ref):
    h_dock = _elu(
        jnp.dot(raw_dock_ref[...], W_pk_ref[...],
                preferred_element_type=jnp.float32) + b_pk_ref[...])
    h_dock_ref[...] = h_dock
    sk = jnp.dot(h_dock, A2_ref[...], preferred_element_type=jnp.float32)
    skT_ref[...] = sk.T.astype(jnp.bfloat16)
    n_dock = h_dock.shape[0]
    for h in range(_NHEADS):
        whk = jnp.dot(h_dock, W_att_ref[h],
                      preferred_element_type=jnp.float32).astype(jnp.bfloat16)
        Whk_ref[h, :, 0:_NHID] = whk
        Whk_ref[h, :, _NHID:_NHID + 1] = jnp.ones((n_dock, 1), jnp.bfloat16)
        Whk_ref[h, :, _NHID + 1:] = jnp.zeros((n_dock, _NHID - 1), jnp.bfloat16)
    h_drone = _elu(
        jnp.dot(raw_drone_ref[...], W_pd_ref[...],
                preferred_element_type=jnp.float32) + b_pd_ref[...])
    sd_ref[...] = jnp.dot(h_drone, A1_ref[...],
                          preferred_element_type=jnp.float32
                          ).astype(jnp.bfloat16)


def _gat_block_kernel(adj_ref, sd_ref, skT_ref, Whk_ref, W_fuse_ref,
                      b_fuse_ref, out_ref):
    maskf = jnp.where(adj_ref[...] > 0, 0.0, _NEG).astype(jnp.bfloat16)
    sd = sd_ref[...]
    acc = jnp.broadcast_to(b_fuse_ref[...], out_ref.shape)
    for h in range(_NHEADS):
        e = sd[:, h:h + 1] + skT_ref[h:h + 1, :]
        e = jnp.maximum(e, jnp.bfloat16(_ALPHA) * e)
        e = e + maskf
        m = jnp.max(e, axis=1, keepdims=True)
        p = jnp.exp(e - m)
        aug = jnp.dot(p, Whk_ref[h],
                      preferred_element_type=jnp.float32)
        s = aug[:, _NHID:_NHID + 1]
        head = _elu(aug[:, 0:_NHID] * (1.0 / s))
        acc = acc + jnp.dot(head, W_fuse_ref[h],
                            preferred_element_type=jnp.float32)
    out_ref[...] = acc


@jax.jit
def kernel(raw_drone, raw_dock, adj, W_pd, b_pd, W_pk, b_pk, W_att, a_att,
           W_fuse, b_fuse):
    n_drone, nfeat_drone = raw_drone.shape
    n_dock, nfeat_dock = raw_dock.shape
    nheads, nhid, _ = W_att.shape

    A1 = jnp.einsum('hij,hj->ih', W_att, a_att[:, :nhid])
    A2 = jnp.einsum('hij,hj->ih', W_att, a_att[:, nhid:])
    W_fuse_h = W_fuse.reshape(nheads, nhid, nhid)
    b_pk2 = b_pk.reshape(1, nhid)
    b_pd2 = b_pd.reshape(1, nhid)
    b_fuse2 = b_fuse.reshape(1, nhid)

    h_dock, Whk, skT, sd = pl.pallas_call(
        _prep_kernel,
        out_shape=(
            jax.ShapeDtypeStruct((n_dock, nhid), jnp.float32),
            jax.ShapeDtypeStruct((nheads, n_dock, 2 * nhid), jnp.bfloat16),
            jax.ShapeDtypeStruct((nheads, n_dock), jnp.bfloat16),
            jax.ShapeDtypeStruct((n_drone, nheads), jnp.bfloat16),
        ),
    )(raw_drone, raw_dock, W_pd, b_pd2, W_pk, b_pk2, W_att, A1, A2)

    blk = 400
    grid = (n_drone // blk,)
    out_drone = pl.pallas_call(
        _gat_block_kernel,
        grid=grid,
        in_specs=[
            pl.BlockSpec((blk, n_dock), lambda i: (i, 0)),
            pl.BlockSpec((blk, nheads), lambda i: (i, 0)),
            pl.BlockSpec((nheads, n_dock), lambda i: (0, 0)),
            pl.BlockSpec((nheads, n_dock, 2 * nhid), lambda i: (0, 0, 0)),
            pl.BlockSpec((nheads, nhid, nhid), lambda i: (0, 0, 0)),
            pl.BlockSpec((1, nhid), lambda i: (0, 0)),
        ],
        out_specs=pl.BlockSpec((blk, nhid), lambda i: (i, 0)),
        out_shape=jax.ShapeDtypeStruct((n_drone, nhid), jnp.float32),
        compiler_params=pltpu.CompilerParams(
            dimension_semantics=("arbitrary",)),
    )(adj, sd, skT, Whk, W_fuse_h, b_fuse2)

    return (out_drone, h_dock)


# speedup vs baseline: 1.8230x; 1.0394x over previous
import jax
import jax.numpy as jnp
from jax.experimental import pallas as pl
from jax.experimental.pallas import tpu as pltpu

_NHEADS = 4
_NHID = 64
_ALPHA = 0.2
_NEG = -9e15


def _elu(x):
    return jnp.where(x > 0, x, jnp.exp(x) - 1.0)


def _prep_kernel(raw_drone_ref, raw_dock_ref, W_pd_ref, b_pd_ref,
                 W_pk_ref, b_pk_ref, W_att_ref, A1_ref, A2_ref,
                 h_dock_ref, Whk_ref, skT_ref, sd_ref):
    h_dock = _elu(
        jnp.dot(raw_dock_ref[...], W_pk_ref[...],
                preferred_element_type=jnp.float32) + b_pk_ref[...])
    h_dock_ref[...] = h_dock
    sk = jnp.dot(h_dock, A2_ref[...], preferred_element_type=jnp.float32)
    skT_ref[...] = sk.T.astype(jnp.bfloat16)
    n_dock = h_dock.shape[0]
    for h in range(_NHEADS):
        whk = jnp.dot(h_dock, W_att_ref[h],
                      preferred_element_type=jnp.float32).astype(jnp.bfloat16)
        Whk_ref[h, :, 0:_NHID] = whk
        Whk_ref[h, :, _NHID:_NHID + 1] = jnp.ones((n_dock, 1), jnp.bfloat16)
        Whk_ref[h, :, _NHID + 1:] = jnp.zeros((n_dock, _NHID - 1), jnp.bfloat16)
    h_drone = _elu(
        jnp.dot(raw_drone_ref[...], W_pd_ref[...],
                preferred_element_type=jnp.float32) + b_pd_ref[...])
    sd_ref[...] = jnp.dot(h_drone, A1_ref[...],
                          preferred_element_type=jnp.float32
                          ).astype(jnp.bfloat16)


def _gat_block_kernel(adj_ref, sd_ref, skT_ref, Whk_ref, W_fuse_ref,
                      b_fuse_ref, out_ref):
    maskf = jnp.where(adj_ref[...] > 0, 0.0, _NEG).astype(jnp.bfloat16)
    sd = sd_ref[...]
    acc = jnp.broadcast_to(b_fuse_ref[...], out_ref.shape)
    for h in range(_NHEADS):
        e = sd[:, h:h + 1] + skT_ref[h:h + 1, :]
        e = jnp.maximum(e, jnp.bfloat16(_ALPHA) * e)
        e = e + maskf
        m = jnp.max(e, axis=1, keepdims=True)
        p = jnp.exp(e - m)
        aug = jnp.dot(p, Whk_ref[h],
                      preferred_element_type=jnp.float32)
        s = aug[:, _NHID:_NHID + 1]
        head = _elu(aug[:, 0:_NHID] * (1.0 / s))
        acc = acc + jnp.dot(head, W_fuse_ref[h],
                            preferred_element_type=jnp.float32)
    out_ref[...] = acc


@jax.jit
def kernel(raw_drone, raw_dock, adj, W_pd, b_pd, W_pk, b_pk, W_att, a_att,
           W_fuse, b_fuse):
    n_drone, nfeat_drone = raw_drone.shape
    n_dock, nfeat_dock = raw_dock.shape
    nheads, nhid, _ = W_att.shape

    A1 = jnp.einsum('hij,hj->ih', W_att, a_att[:, :nhid])
    A2 = jnp.einsum('hij,hj->ih', W_att, a_att[:, nhid:])
    W_fuse_h = W_fuse.reshape(nheads, nhid, nhid)
    b_pk2 = b_pk.reshape(1, nhid)
    b_pd2 = b_pd.reshape(1, nhid)
    b_fuse2 = b_fuse.reshape(1, nhid)

    h_dock, Whk, skT, sd = pl.pallas_call(
        _prep_kernel,
        out_shape=(
            jax.ShapeDtypeStruct((n_dock, nhid), jnp.float32),
            jax.ShapeDtypeStruct((nheads, n_dock, 2 * nhid), jnp.bfloat16),
            jax.ShapeDtypeStruct((nheads, n_dock), jnp.bfloat16),
            jax.ShapeDtypeStruct((n_drone, nheads), jnp.bfloat16),
        ),
    )(raw_drone, raw_dock, W_pd, b_pd2, W_pk, b_pk2, W_att, A1, A2)

    blk = 1000
    grid = (n_drone // blk,)
    out_drone = pl.pallas_call(
        _gat_block_kernel,
        grid=grid,
        in_specs=[
            pl.BlockSpec((blk, n_dock), lambda i: (i, 0)),
            pl.BlockSpec((blk, nheads), lambda i: (i, 0)),
            pl.BlockSpec((nheads, n_dock), lambda i: (0, 0)),
            pl.BlockSpec((nheads, n_dock, 2 * nhid), lambda i: (0, 0, 0)),
            pl.BlockSpec((nheads, nhid, nhid), lambda i: (0, 0, 0)),
            pl.BlockSpec((1, nhid), lambda i: (0, 0)),
        ],
        out_specs=pl.BlockSpec((blk, nhid), lambda i: (i, 0)),
        out_shape=jax.ShapeDtypeStruct((n_drone, nhid), jnp.float32),
        compiler_params=pltpu.CompilerParams(
            dimension_semantics=("arbitrary",)),
    )(adj, sd, skT, Whk, W_fuse_h, b_fuse2)

    return (out_drone, h_dock)
